# baseline (speedup 1.0000x reference)
import functools

import jax
import jax.numpy as jnp
from jax import lax
from jax.experimental import pallas as pl
from jax.experimental.pallas import tpu as pltpu
from jax.experimental.pallas import tpu_sc as plsc

N = 5000
NP = 5120
K = 100
NEG = -1e30
IOU_T = 0.25
CONF_T = 0.5
IMG = 512.0
NW = 16
EPW = NP // NW
NV = EPW // 16

_mesh = plsc.VectorSubcoreMesh(core_axis_name="c", subcore_axis_name="s")


def _bcast(x, dtype=jnp.float32):
    return jnp.broadcast_to(jnp.asarray(x, dtype), (16,))


@functools.partial(
    pl.kernel,
    out_type=jax.ShapeDtypeStruct((600,), jnp.float32),
    mesh=_mesh,
    scratch_types=[
        pltpu.VMEM((22 * EPW,), jnp.float32),
        pltpu.VMEM((EPW,), jnp.float32),
        pltpu.VMEM((EPW,), jnp.float32),
        pltpu.VMEM((EPW,), jnp.float32),
        pltpu.VMEM((EPW,), jnp.float32),
        pltpu.VMEM((EPW,), jnp.float32),
        pltpu.VMEM((EPW,), jnp.float32),
        pltpu.VMEM((EPW,), jnp.float32),
        pltpu.VMEM((16,), jnp.float32),
        pltpu.VMEM((NW * 16,), jnp.float32),
        pltpu.VMEM_SHARED((2 * NW * 16,), jnp.float32),
        pltpu.VMEM((6 * 128,), jnp.float32),
        pltpu.VMEM((128,), jnp.int32),
        pltpu.SMEM((16,), jnp.int32),
        pltpu.VMEM((640,), jnp.float32),
    ],
    compiler_params=pltpu.CompilerParams(needs_layout_passes=False),
)
def _sc_post(x_hbm, o_hbm, xv, sv, y1m, x1m, y2m, x2m, am, clm,
             rec, allm, shared, selbuf, posv, offm, outm):
    s_id = lax.axis_index("s")
    c_id = lax.axis_index("c")
    base = s_id * EPW
    iota = lax.broadcasted_iota(jnp.int32, (16,), 0)
    lane0 = iota == 0

    pltpu.sync_copy(x_hbm.at[pl.ds(s_id * 22 * EPW, 22 * EPW)], xv)

    for j in range(NV):
        xc = lambda c: xv[pl.ds(c * EPW + j * 16, 16)]
        sl = pl.ds(j * 16, 16)
        conf = xc(0)
        clsf = jnp.zeros((16,), jnp.float32)
        for c in range(1, 10):
            scc = xc(c)
            take = scc > conf
            conf = jnp.where(take, scc, conf)
            clsf = jnp.where(take, jnp.float32(c), clsf)
        c10, c11, c12, c13 = xc(10), xc(11), xc(12), xc(13)
        c14, c15, c16, c17 = xc(14), xc(15), xc(16), xc(17)
        c18, c19, c20, c21 = xc(18), xc(19), xc(20), xc(21)
        pred41 = jnp.exp(c12 * c20)
        pred51 = jnp.exp(c13 * c21)
        w = pred41 + pred41 * c16
        h = pred51 + pred51 * c17
        cx = c14 + c10 * c18 * c16
        cy = c15 + c11 * c19 * c17
        x1v = jnp.maximum(cx - w / 2.0, 0.0)
        x2v = jnp.minimum(cx + w / 2.0, IMG - 1.0)
        y1v = jnp.maximum(cy - h / 2.0, 0.0)
        y2v = jnp.minimum(cy + h / 2.0, IMG - 1.0)
        gidx = base + j * 16 + iota
        valid = (clsf != 0.0) & ((conf - CONF_T) != 0.0) & (gidx < N)
        sv[sl] = jnp.where(valid, conf, NEG)
        y1m[sl] = y1v
        x1m[sl] = x1v
        y2m[sl] = y2v
        x2m[sl] = x2v
        am[sl] = (y2v - y1v) * (x2v - x1v)
        clm[sl] = clsf

    def cond(carry):
        k, ok = carry
        return (k < K) & ok

    def body(carry):
        k, _ = carry
        bestv = _bcast(-3e38)
        bestj = _bcast(0, jnp.int32)
        for j in range(NV):
            v = sv[pl.ds(j * 16, 16)]
            upd = v > bestv
            bestv = jnp.where(upd, v, bestv)
            bestj = jnp.where(upd, j, bestj)
        bm = jnp.max(bestv)
        lidx = jnp.min(jnp.where(bestv == bm, bestj * 16 + iota,
                                 jnp.int32(1 << 30)))
        lv = _bcast(lidx, jnp.int32)
        ext = lambda ref: plsc.load_gather(ref, [lv])[0]

        recv = _bcast(0.0)
        recv = jnp.where(iota == 0, bm, recv)
        recv = jnp.where(iota == 1, ext(y1m), recv)
        recv = jnp.where(iota == 2, ext(x1m), recv)
        recv = jnp.where(iota == 3, ext(y2m), recv)
        recv = jnp.where(iota == 4, ext(x2m), recv)
        recv = jnp.where(iota == 5, ext(am), recv)
        recv = jnp.where(iota == 6, ext(clm), recv)
        rec[...] = recv
        buf = (k % 2) * (NW * 16)
        pltpu.sync_copy(rec, shared.at[pl.ds(buf + s_id * 16, 16)])
        plsc.subcore_barrier()
        pltpu.sync_copy(shared.at[pl.ds(buf, NW * 16)], allm)

        svec = plsc.load_gather(allm, [iota * 16])
        gm = jnp.max(svec)
        wv = jnp.min(jnp.where(svec == gm, iota, 99))
        ok = gm > (NEG / 2)
        wrec = allm[pl.ds(wv * 16, 16)]
        gy1 = wrec[1]
        gx1 = wrec[2]
        gy2 = wrec[3]
        gx2 = wrec[4]
        ga = wrec[5]
        gc = wrec[6]

        for j in range(NV):
            sl = pl.ds(j * 16, 16)
            y1v = y1m[sl]
            x1v = x1m[sl]
            y2v = y2m[sl]
            x2v = x2m[sl]
            yy1 = jnp.maximum(gy1, y1v)
            xx1 = jnp.maximum(gx1, x1v)
            yy2 = jnp.minimum(gy2, y2v)
            xx2 = jnp.minimum(gx2, x2v)
            inter = (jnp.maximum(yy2 - yy1, 0.0)
                     * jnp.maximum(xx2 - xx1, 0.0))
            iou = inter / (ga + am[sl] - inter + 1e-12)
            sv[sl] = jnp.where(ok & (iou > IOU_T), NEG, sv[sl])

        plsc.store_scatter(sv, [_bcast(lidx, jnp.int32)],
                           _bcast(NEG), mask=lane0 & (wv == s_id))

        val = _bcast(0.0)
        val = jnp.where(iota == 0, gc, val)
        val = jnp.where(iota == 1, gm, val)
        val = jnp.where(iota == 2, gy1, val)
        val = jnp.where(iota == 3, gx1, val)
        val = jnp.where(iota == 4, gy2, val)
        val = jnp.where(iota == 5, gx2, val)
        plsc.store_scatter(selbuf, [iota * 128 + k], val,
                           mask=(iota < 6) & ok)

        return (jnp.where(ok, k + 1, k), ok)

    kf, _unused = lax.while_loop(cond, body, (jnp.int32(0), jnp.bool_(True)))

    for c in range(16):
        offm[c] = jnp.int32(0)

    def _sel_cls(i):
        chunk = (i // 16) * 16
        v = selbuf[pl.ds(chunk, 16)]
        c = jnp.sum(jnp.where(iota == i - chunk, v, 0.0))
        return c.astype(jnp.int32)

    def cbody(i, acc):
        c = _sel_cls(i)
        offm[c] = offm[c] + 1
        return acc

    lax.fori_loop(0, kf, cbody, jnp.int32(0))

    def pbody(c, run):
        t = offm[c]
        offm[c] = run
        return run + t

    lax.fori_loop(1, 10, pbody, jnp.int32(0))

    def obody(i, acc):
        ci = _sel_cls(i)
        p = offm[ci]
        offm[ci] = p + 1
        plsc.store_scatter(posv, [_bcast(i, jnp.int32)],
                           _bcast(p, jnp.int32), mask=lane0)
        return acc

    lax.fori_loop(0, kf, obody, jnp.int32(0))

    for j in range(40):
        outm[pl.ds(j * 16, 16)] = jnp.zeros((16,), jnp.float32)

    for j in range(8):
        sl = pl.ds(j * 16, 16)
        pv = posv[sl]
        okm = (iota + j * 16) < kf
        clsv = selbuf[pl.ds(0 * 128 + j * 16, 16)]
        scv = selbuf[pl.ds(1 * 128 + j * 16, 16)]
        ny1 = selbuf[pl.ds(2 * 128 + j * 16, 16)] / IMG
        nx1 = selbuf[pl.ds(3 * 128 + j * 16, 16)] / IMG
        ny2 = 1.0 - selbuf[pl.ds(4 * 128 + j * 16, 16)] / IMG
        nx2 = 1.0 - selbuf[pl.ds(5 * 128 + j * 16, 16)] / IMG
        pb = pv * 6
        plsc.store_scatter(outm, [pb], clsv, mask=okm)
        plsc.store_scatter(outm, [pb + 1], scv, mask=okm)
        plsc.store_scatter(outm, [pb + 2], ny1, mask=okm)
        plsc.store_scatter(outm, [pb + 3], nx1, mask=okm)
        plsc.store_scatter(outm, [pb + 4], ny2, mask=okm)
        plsc.store_scatter(outm, [pb + 5], nx2, mask=okm)

    @pl.when((c_id == 0) & (s_id == 0))
    def _():
        pltpu.sync_copy(outm.at[pl.ds(0, 600)], o_hbm)


def kernel(x):
    x0 = jnp.transpose(x[0])
    x22 = jnp.pad(x0, ((0, 0), (0, NP - N)))
    xw = jnp.transpose(x22.reshape(22, NW, EPW), (1, 0, 2)).ravel()
    out = _sc_post(xw)
    return jnp.broadcast_to(out.reshape(1, K, 6), (x.shape[0], K, 6))


# speedup vs baseline: 1.0066x; 1.0066x over previous
import functools

import jax
import jax.numpy as jnp
from jax import lax
from jax.experimental import pallas as pl
from jax.experimental.pallas import tpu as pltpu
from jax.experimental.pallas import tpu_sc as plsc

N = 5000
NP = 5120
K = 100
NEG = -1e30
IOU_T = 0.25
CONF_T = 0.5
IMG = 512.0
NW = 16
EPW = NP // NW
NV = EPW // 16

_mesh = plsc.VectorSubcoreMesh(core_axis_name="c", subcore_axis_name="s")


def _bcast(x, dtype=jnp.float32):
    return jnp.broadcast_to(jnp.asarray(x, dtype), (16,))


@functools.partial(
    pl.kernel,
    out_type=jax.ShapeDtypeStruct((600,), jnp.float32),
    mesh=_mesh,
    scratch_types=[
        pltpu.VMEM((22 * EPW,), jnp.float32),
        pltpu.VMEM((EPW,), jnp.float32),
        pltpu.VMEM((EPW,), jnp.float32),
        pltpu.VMEM((EPW,), jnp.float32),
        pltpu.VMEM((EPW,), jnp.float32),
        pltpu.VMEM((EPW,), jnp.float32),
        pltpu.VMEM((EPW,), jnp.float32),
        pltpu.VMEM((EPW,), jnp.float32),
        pltpu.VMEM((16,), jnp.float32),
        pltpu.VMEM((NW * 16,), jnp.float32),
        pltpu.VMEM_SHARED((2 * NW * 16,), jnp.float32),
        pltpu.VMEM((6 * 128,), jnp.float32),
        pltpu.VMEM((128,), jnp.int32),
        pltpu.SMEM((16,), jnp.int32),
        pltpu.SMEM((128,), jnp.int32),
        pltpu.VMEM((640,), jnp.float32),
    ],
    compiler_params=pltpu.CompilerParams(needs_layout_passes=False),
)
def _sc_post(x_hbm, o_hbm, xv, sv, y1m, x1m, y2m, x2m, am, clm,
             rec, allm, shared, selbuf, posv, offm, clsm, outm):
    s_id = lax.axis_index("s")
    c_id = lax.axis_index("c")
    base = s_id * EPW
    iota = lax.broadcasted_iota(jnp.int32, (16,), 0)
    lane0 = iota == 0

    pltpu.sync_copy(x_hbm.at[pl.ds(s_id * 22 * EPW, 22 * EPW)], xv)

    for j in range(NV):
        xc = lambda c: xv[pl.ds(c * EPW + j * 16, 16)]
        sl = pl.ds(j * 16, 16)
        conf = xc(0)
        clsf = jnp.zeros((16,), jnp.float32)
        for c in range(1, 10):
            scc = xc(c)
            take = scc > conf
            conf = jnp.where(take, scc, conf)
            clsf = jnp.where(take, jnp.float32(c), clsf)
        c10, c11, c12, c13 = xc(10), xc(11), xc(12), xc(13)
        c14, c15, c16, c17 = xc(14), xc(15), xc(16), xc(17)
        c18, c19, c20, c21 = xc(18), xc(19), xc(20), xc(21)
        pred41 = jnp.exp(c12 * c20)
        pred51 = jnp.exp(c13 * c21)
        w = pred41 + pred41 * c16
        h = pred51 + pred51 * c17
        cx = c14 + c10 * c18 * c16
        cy = c15 + c11 * c19 * c17
        x1v = jnp.maximum(cx - w / 2.0, 0.0)
        x2v = jnp.minimum(cx + w / 2.0, IMG - 1.0)
        y1v = jnp.maximum(cy - h / 2.0, 0.0)
        y2v = jnp.minimum(cy + h / 2.0, IMG - 1.0)
        gidx = base + j * 16 + iota
        valid = (clsf != 0.0) & ((conf - CONF_T) != 0.0) & (gidx < N)
        sv[sl] = jnp.where(valid, conf, NEG)
        y1m[sl] = y1v
        x1m[sl] = x1v
        y2m[sl] = y2v
        x2m[sl] = x2v
        am[sl] = (y2v - y1v) * (x2v - x1v)
        clm[sl] = clsf

    def cond(carry):
        k, ok = carry
        return (k < K) & ok

    def body(carry):
        k, _ = carry
        bestv = _bcast(-3e38)
        bestj = _bcast(0, jnp.int32)
        for j in range(NV):
            v = sv[pl.ds(j * 16, 16)]
            upd = v > bestv
            bestv = jnp.where(upd, v, bestv)
            bestj = jnp.where(upd, j, bestj)
        bm = jnp.max(bestv)
        lidx = jnp.min(jnp.where(bestv == bm, bestj * 16 + iota,
                                 jnp.int32(1 << 30)))
        lv = _bcast(lidx, jnp.int32)
        ext = lambda ref: plsc.load_gather(ref, [lv])[0]

        recv = _bcast(0.0)
        recv = jnp.where(iota == 0, bm, recv)
        recv = jnp.where(iota == 1, ext(y1m), recv)
        recv = jnp.where(iota == 2, ext(x1m), recv)
        recv = jnp.where(iota == 3, ext(y2m), recv)
        recv = jnp.where(iota == 4, ext(x2m), recv)
        recv = jnp.where(iota == 5, ext(am), recv)
        recv = jnp.where(iota == 6, ext(clm), recv)
        rec[...] = recv
        buf = (k % 2) * (NW * 16)
        pltpu.sync_copy(rec, shared.at[pl.ds(buf + s_id * 16, 16)])
        plsc.subcore_barrier()
        pltpu.sync_copy(shared.at[pl.ds(buf, NW * 16)], allm)

        svec = plsc.load_gather(allm, [iota * 16])
        gm = jnp.max(svec)
        wv = jnp.min(jnp.where(svec == gm, iota, 99))
        ok = gm > (NEG / 2)
        wrec = allm[pl.ds(wv * 16, 16)]
        gy1 = wrec[1]
        gx1 = wrec[2]
        gy2 = wrec[3]
        gx2 = wrec[4]
        ga = wrec[5]
        gc = wrec[6]

        for j in range(NV):
            sl = pl.ds(j * 16, 16)
            y1v = y1m[sl]
            x1v = x1m[sl]
            y2v = y2m[sl]
            x2v = x2m[sl]
            yy1 = jnp.maximum(gy1, y1v)
            xx1 = jnp.maximum(gx1, x1v)
            yy2 = jnp.minimum(gy2, y2v)
            xx2 = jnp.minimum(gx2, x2v)
            inter = (jnp.maximum(yy2 - yy1, 0.0)
                     * jnp.maximum(xx2 - xx1, 0.0))
            iou = inter / (ga + am[sl] - inter + 1e-12)
            sv[sl] = jnp.where(ok & (iou > IOU_T), NEG, sv[sl])

        plsc.store_scatter(sv, [_bcast(lidx, jnp.int32)],
                           _bcast(NEG), mask=lane0 & (wv == s_id))

        val = _bcast(0.0)
        val = jnp.where(iota == 0, gc, val)
        val = jnp.where(iota == 1, gm, val)
        val = jnp.where(iota == 2, gy1, val)
        val = jnp.where(iota == 3, gx1, val)
        val = jnp.where(iota == 4, gy2, val)
        val = jnp.where(iota == 5, gx2, val)
        plsc.store_scatter(selbuf, [iota * 128 + k], val,
                           mask=(iota < 6) & ok)
        clsm[k] = gc.astype(jnp.int32)

        return (jnp.where(ok, k + 1, k), ok)

    kf, _unused = lax.while_loop(cond, body, (jnp.int32(0), jnp.bool_(True)))

    for c in range(16):
        offm[c] = jnp.int32(0)

    def _sel_cls(i):
        chunk = (i // 16) * 16
        v = selbuf[pl.ds(chunk, 16)]
        c = jnp.sum(jnp.where(iota == i - chunk, v, 0.0))
        return c.astype(jnp.int32)

    def cbody(i, acc):
        c = _sel_cls(i)
        offm[c] = offm[c] + 1
        return acc

    lax.fori_loop(0, kf, cbody, jnp.int32(0))

    def pbody(c, run):
        t = offm[c]
        offm[c] = run
        return run + t

    lax.fori_loop(1, 10, pbody, jnp.int32(0))

    def obody(i, acc):
        ci = clsm[i]
        p = offm[ci]
        offm[ci] = p + 1
        plsc.store_scatter(posv, [_bcast(i, jnp.int32)],
                           _bcast(p, jnp.int32), mask=lane0)
        return acc

    lax.fori_loop(0, kf, obody, jnp.int32(0))

    for j in range(40):
        outm[pl.ds(j * 16, 16)] = jnp.zeros((16,), jnp.float32)

    for j in range(8):
        sl = pl.ds(j * 16, 16)
        pv = posv[sl]
        okm = (iota + j * 16) < kf
        clsv = selbuf[pl.ds(0 * 128 + j * 16, 16)]
        scv = selbuf[pl.ds(1 * 128 + j * 16, 16)]
        ny1 = selbuf[pl.ds(2 * 128 + j * 16, 16)] / IMG
        nx1 = selbuf[pl.ds(3 * 128 + j * 16, 16)] / IMG
        ny2 = 1.0 - selbuf[pl.ds(4 * 128 + j * 16, 16)] / IMG
        nx2 = 1.0 - selbuf[pl.ds(5 * 128 + j * 16, 16)] / IMG
        pb = pv * 6
        plsc.store_scatter(outm, [pb], clsv, mask=okm)
        plsc.store_scatter(outm, [pb + 1], scv, mask=okm)
        plsc.store_scatter(outm, [pb + 2], ny1, mask=okm)
        plsc.store_scatter(outm, [pb + 3], nx1, mask=okm)
        plsc.store_scatter(outm, [pb + 4], ny2, mask=okm)
        plsc.store_scatter(outm, [pb + 5], nx2, mask=okm)

    @pl.when((c_id == 0) & (s_id == 0))
    def _():
        pltpu.sync_copy(outm.at[pl.ds(0, 600)], o_hbm)


def kernel(x):
    x0 = jnp.transpose(x[0])
    x22 = jnp.pad(x0, ((0, 0), (0, NP - N)))
    xw = jnp.transpose(x22.reshape(22, NW, EPW), (1, 0, 2)).ravel()
    out = _sc_post(xw)
    return jnp.broadcast_to(out.reshape(1, K, 6), (x.shape[0], K, 6))
